# baseline (speedup 1.0000x reference)
import jax
import jax.numpy as jnp
from jax.experimental import pallas as pl
from jax.experimental.pallas import tpu as pltpu


def _stats_kernel(w_ref, k_ref, out_ref):
    wv = w_ref[...]
    m = jnp.max(wv)
    d = jnp.sum(jnp.exp(wv - m))
    coeff = jnp.round(k_ref[0, 0]) / (jnp.float32(wv.size) * d)
    out_ref[...] = jnp.stack([m, coeff]).reshape(1, 2)


def _wsum_packed_kernel(t, stats_ref, w_ref, x_ref, out_ref, acc_ref):
    i = pl.program_id(0)
    m = stats_ref[0, 0]
    coeff = stats_ref[0, 1]
    e2 = jnp.exp(w_ref[0] - m) * coeff
    part = jax.lax.dot_general(
        e2, x_ref[...], (((1,), (0,)), ((), ())),
        preferred_element_type=jnp.float32)

    @pl.when(i == 0)
    def _init():
        acc_ref[...] = jnp.zeros_like(acc_ref)

    acc_ref[...] += part

    @pl.when(i == pl.num_programs(0) - 1)
    def _fold():
        a = acc_ref[...]
        out_ref[...] = a[0:1, 0:t] + a[1:2, t:2 * t]


def _wsum_plain_kernel(stats_ref, w_ref, x_ref, out_ref):
    i = pl.program_id(0)
    m = stats_ref[0, 0]
    coeff = stats_ref[0, 1]
    e = jnp.exp(w_ref[0] - m) * coeff
    part = jax.lax.dot_general(
        e, x_ref[...], (((1,), (0,)), ((), ())),
        preferred_element_type=jnp.float32)

    @pl.when(i == 0)
    def _init():
        out_ref[...] = jnp.zeros_like(out_ref)

    out_ref[...] += part


def _pick_block(n):
    for b in (40000, 10000, 8000, 5000, 4096, 4000, 2048, 2000, 1000):
        if n % b == 0:
            return b
    return n


def kernel(x, w, k_param):
    n, t = x.shape
    b = _pick_block(n)
    g = n // b
    rows = 1000 if n % 1000 == 0 else 1

    stats = pl.pallas_call(
        _stats_kernel,
        out_shape=jax.ShapeDtypeStruct((1, 2), jnp.float32),
        in_specs=[
            pl.BlockSpec((n // rows, rows), lambda: (0, 0)),
            pl.BlockSpec((1, 1), lambda: (0, 0)),
        ],
        out_specs=pl.BlockSpec((1, 2), lambda: (0, 0)),
    )(w.reshape(n // rows, rows), k_param.reshape(1, 1))

    packed = (2 * t == 128) and (b % 16 == 0)
    if packed:
        b2 = b // 2
        w2 = w.reshape(g, b2, 2).swapaxes(1, 2)
        x2 = x.reshape(n // 2, 2 * t)
        out = pl.pallas_call(
            lambda *refs: _wsum_packed_kernel(t, *refs),
            grid=(g,),
            out_shape=jax.ShapeDtypeStruct((1, t), jnp.float32),
            in_specs=[
                pl.BlockSpec((1, 2), lambda i: (0, 0)),
                pl.BlockSpec((1, 2, b2), lambda i: (i, 0, 0)),
                pl.BlockSpec((b2, 2 * t), lambda i: (i, 0)),
            ],
            out_specs=pl.BlockSpec((1, t), lambda i: (0, 0)),
            scratch_shapes=[pltpu.VMEM((2, 2 * t), jnp.float32)],
        )(stats, w2, x2)
    else:
        out = pl.pallas_call(
            _wsum_plain_kernel,
            grid=(g,),
            out_shape=jax.ShapeDtypeStruct((1, t), jnp.float32),
            in_specs=[
                pl.BlockSpec((1, 2), lambda i: (0, 0)),
                pl.BlockSpec((1, 1, b), lambda i: (i, 0, 0)),
                pl.BlockSpec((b, t), lambda i: (i, 0)),
            ],
            out_specs=pl.BlockSpec((1, t), lambda i: (0, 0)),
        )(stats, w.reshape(g, 1, b), x)

    return out.reshape(t)


# speedup vs baseline: 2.0208x; 2.0208x over previous
import jax
import jax.numpy as jnp
from jax.experimental import pallas as pl


def _stats_kernel(w_ref, k_ref, out_ref):
    wv = w_ref[...]
    m = jnp.max(wv)
    d = jnp.sum(jnp.exp(wv - m))
    coeff = jnp.round(k_ref[0, 0]) / (jnp.float32(wv.size) * d)
    out_ref[...] = jnp.stack([m, coeff]).reshape(1, 2)


def _wsum_multi_kernel(s, stats_ref, w_ref, *rest):
    x_refs = rest[:s]
    out_ref = rest[s]
    i = pl.program_id(0)
    m = stats_ref[0, 0]
    coeff = stats_ref[0, 1]
    e = jnp.exp(w_ref[0] - m) * coeff
    part = jax.lax.dot_general(
        e[0:1], x_refs[0][...], (((1,), (0,)), ((), ())),
        preferred_element_type=jnp.float32)
    for j in range(1, s):
        part += jax.lax.dot_general(
            e[j:j + 1], x_refs[j][...], (((1,), (0,)), ((), ())),
            preferred_element_type=jnp.float32)

    @pl.when(i == 0)
    def _init():
        out_ref[...] = jnp.zeros_like(out_ref)

    out_ref[...] += part


def _pick_split(n):
    for s, b in ((4, 10000), (4, 5000), (2, 10000), (1, 10000), (1, 8000),
                 (1, 5000), (1, 4096), (1, 4000), (1, 2048), (1, 2000),
                 (1, 1000)):
        if n % (s * b) == 0:
            return s, b
    return 1, n


def kernel(x, w, k_param):
    n, t = x.shape
    s, b = _pick_split(n)
    g2 = n // (s * b)
    rows = 1000 if n % 1000 == 0 else 1

    stats = pl.pallas_call(
        _stats_kernel,
        out_shape=jax.ShapeDtypeStruct((1, 2), jnp.float32),
        in_specs=[
            pl.BlockSpec((n // rows, rows), lambda: (0, 0)),
            pl.BlockSpec((1, 1), lambda: (0, 0)),
        ],
        out_specs=pl.BlockSpec((1, 2), lambda: (0, 0)),
    )(w.reshape(n // rows, rows), k_param.reshape(1, 1))

    w2 = w.reshape(s, g2, b).swapaxes(0, 1)
    x_specs = [
        pl.BlockSpec((b, t), lambda i, j=j: (j * g2 + i, 0)) for j in range(s)
    ]
    out = pl.pallas_call(
        lambda *refs: _wsum_multi_kernel(s, *refs),
        grid=(g2,),
        out_shape=jax.ShapeDtypeStruct((1, t), jnp.float32),
        in_specs=[
            pl.BlockSpec((1, 2), lambda i: (0, 0)),
            pl.BlockSpec((1, s, b), lambda i: (i, 0, 0)),
            *x_specs,
        ],
        out_specs=pl.BlockSpec((1, t), lambda i: (0, 0)),
    )(stats, w2, *([x] * s))

    return out.reshape(t)
